# Initial kernel scaffold; baseline (speedup 1.0000x reference)
import jax
import jax.numpy as jnp
from jax.experimental import pallas as pl


def kernel(X, enc_W, enc_b, M1_W, M2_W, U_W, mst_W, mst_b, pred_W1, pred_b1, pred_W2, pred_b2):
    raise NotImplementedError("write your pallas kernel here")


# speedup vs baseline: 27.1836x; 27.1836x over previous
import jax
import jax.numpy as jnp
from jax.experimental import pallas as pl

_N = 48
_L = 64
_STEPS = _N - 1


def _leaky(x):
    return jnp.where(x >= 0, x, 0.01 * x)


def _prims_kernel(x0c_ref, x1c_ref, x0r_ref, x1r_ref,
                  enc_w0_ref, enc_w1_ref, enc_b_ref,
                  m1d_ref, m1s_ref, m1v_ref,
                  m2w_ref, u1_ref, u2_ref,
                  mw0_ref, mw1_ref, mb_ref,
                  p1a_ref, p1b_ref, p1c_ref, p1d_ref, pb1_ref,
                  p2_ref, pb2_ref,
                  out_ref):
    d0 = x0c_ref[:, :] - x0r_ref[:, :]
    d1 = x1c_ref[:, :] - x1r_ref[:, :]
    ew = jnp.sqrt(d0 * d0 + d1 * d1 + 1e-12)
    ewv = ew[:, :, None] * m1v_ref[:, :][None, :, :]

    enc_w0 = enc_w0_ref[:, :]
    enc_w1 = enc_w1_ref[:, :]
    enc_b = enc_b_ref[:, :]
    m1d = m1d_ref[:, :]
    m1s = m1s_ref[:, :]
    m2w = m2w_ref[:, :]
    u1 = u1_ref[:, :]
    u2 = u2_ref[:, :]
    mw0 = mw0_ref[:, :]
    mw1 = mw1_ref[:, :]
    mb = mb_ref[:, :]

    iota = jax.lax.broadcasted_iota(jnp.int32, (_N, 1), 0)

    def step(_, carry):
        h, pt, _enc = carry
        encoded = jnp.maximum(pt * enc_w0 + h @ enc_w1 + enc_b, 0.0)
        srcp = encoded @ m1s
        dstp = encoded @ m1d
        m1 = _leaky(srcp[:, None, :] + dstp[None, :, :] + ewv)
        m2 = _leaky(m1.reshape(_N * _N, _L) @ m2w)
        aggr = jnp.max(m2.reshape(_N, _N, _L), axis=0)
        h_new = jnp.clip(_leaky(encoded @ u1 + aggr @ u2), -1e9, 1e9)
        logits = jax.nn.sigmoid(encoded @ mw0 + h_new @ mw1 + mb)
        mx = jnp.max(logits)
        idx = jnp.min(jnp.where(logits == mx, iota, _N))
        pt_new = jnp.where(iota == idx, 1.0, pt)
        return (h_new, pt_new, encoded)

    init = (jnp.zeros((_N, _L), jnp.float32),
            jnp.zeros((_N, 1), jnp.float32),
            jnp.zeros((_N, _L), jnp.float32))
    h, _pt, enc = jax.lax.fori_loop(0, _STEPS, step, init)

    s_part = enc @ p1a_ref[:, :] + h @ p1b_ref[:, :]
    d_part = enc @ p1c_ref[:, :] + h @ p1d_ref[:, :]
    pe = jnp.maximum(
        s_part[:, None, :] + d_part[None, :, :] + pb1_ref[:, :][None, :, :],
        0.0)
    out_ref[:, :] = pe.reshape(_N * _N, _L) @ p2_ref[:, :] + pb2_ref[:, :]


def kernel(X, enc_W, enc_b, M1_W, M2_W, U_W, mst_W, mst_b,
           pred_W1, pred_b1, pred_W2, pred_b2):
    x0c = X[:, 0:1]
    x1c = X[:, 1:2]
    x0r = X[:, 0].reshape(1, _N)
    x1r = X[:, 1].reshape(1, _N)
    args = (
        x0c, x1c, x0r, x1r,
        enc_W[0:1, :], enc_W[1:, :], enc_b.reshape(1, _L),
        M1_W[0:_L, :], M1_W[_L:2 * _L, :], M1_W[2 * _L:, :],
        M2_W,
        U_W[0:_L, :], U_W[_L:, :],
        mst_W[0:_L, :], mst_W[_L:, :], mst_b.reshape(1, 1),
        pred_W1[0:_L, :], pred_W1[_L:2 * _L, :],
        pred_W1[2 * _L:3 * _L, :], pred_W1[3 * _L:, :],
        pred_b1.reshape(1, _L),
        pred_W2, pred_b2.reshape(1, 1),
    )
    out = pl.pallas_call(
        _prims_kernel,
        out_shape=jax.ShapeDtypeStruct((_N * _N, 1), jnp.float32),
    )(*args)
    return out.reshape(_N, _N)


# speedup vs baseline: 28.0302x; 1.0311x over previous
import jax
import jax.numpy as jnp
from jax.experimental import pallas as pl

_N = 48
_L = 64
_STEPS = _N - 1


def _leaky(x):
    return jnp.maximum(x, 0.01 * x)


def _prims_kernel(x0c_ref, x1c_ref, x0r_ref, x1r_ref,
                  enc_w0_ref, enc_w1_ref, enc_b_ref,
                  m1d_ref, m1s_ref, m1v_ref,
                  m2w_ref, u1_ref, u2_ref,
                  mw0_ref, mw1_ref, mb_ref,
                  p1a_ref, p1b_ref, p1c_ref, p1d_ref, pb1_ref,
                  p2_ref, pb2_ref,
                  out_ref):
    d0 = x0c_ref[:, :] - x0r_ref[:, :]
    d1 = x1c_ref[:, :] - x1r_ref[:, :]
    ew = jnp.sqrt(d0 * d0 + d1 * d1 + 1e-12)
    ewv = ew[:, :, None] * m1v_ref[:, :][None, :, :]

    enc_w0 = enc_w0_ref[:, :]
    enc_w1 = enc_w1_ref[:, :]
    enc_b = enc_b_ref[:, :]
    m1d = m1d_ref[:, :]
    m1s = m1s_ref[:, :]
    m2w = m2w_ref[:, :]
    u1 = u1_ref[:, :]
    u2 = u2_ref[:, :]
    mw0 = mw0_ref[:, :]
    mw1 = mw1_ref[:, :]
    mb = mb_ref[:, :]

    iota = jax.lax.broadcasted_iota(jnp.int32, (_N, 1), 0)

    def step(_, carry):
        h, pt, _enc = carry
        encoded = jnp.maximum(pt * enc_w0 + h @ enc_w1 + enc_b, 0.0)
        srcp = encoded @ m1s
        dstp = encoded @ m1d
        m1 = _leaky(srcp[:, None, :] + dstp[None, :, :] + ewv)
        z = m1.reshape(_N * _N, _L) @ m2w
        aggr = _leaky(jnp.max(z.reshape(_N, _N, _L), axis=0))
        h_new = jnp.clip(_leaky(encoded @ u1 + aggr @ u2), -1e9, 1e9)
        logits = jax.nn.sigmoid(encoded @ mw0 + h_new @ mw1 + mb)
        mx = jnp.max(logits)
        idx = jnp.min(jnp.where(logits == mx, iota, _N))
        pt_new = jnp.where(iota == idx, 1.0, pt)
        return (h_new, pt_new, encoded)

    init = (jnp.zeros((_N, _L), jnp.float32),
            jnp.zeros((_N, 1), jnp.float32),
            jnp.zeros((_N, _L), jnp.float32))
    h, _pt, enc = jax.lax.fori_loop(0, _STEPS, step, init)

    s_part = enc @ p1a_ref[:, :] + h @ p1b_ref[:, :]
    d_part = enc @ p1c_ref[:, :] + h @ p1d_ref[:, :]
    pe = jnp.maximum(
        s_part[:, None, :] + d_part[None, :, :] + pb1_ref[:, :][None, :, :],
        0.0)
    out_ref[:, :] = pe.reshape(_N * _N, _L) @ p2_ref[:, :] + pb2_ref[:, :]


def kernel(X, enc_W, enc_b, M1_W, M2_W, U_W, mst_W, mst_b,
           pred_W1, pred_b1, pred_W2, pred_b2):
    x0c = X[:, 0:1]
    x1c = X[:, 1:2]
    x0r = X[:, 0].reshape(1, _N)
    x1r = X[:, 1].reshape(1, _N)
    args = (
        x0c, x1c, x0r, x1r,
        enc_W[0:1, :], enc_W[1:, :], enc_b.reshape(1, _L),
        M1_W[0:_L, :], M1_W[_L:2 * _L, :], M1_W[2 * _L:, :],
        M2_W,
        U_W[0:_L, :], U_W[_L:, :],
        mst_W[0:_L, :], mst_W[_L:, :], mst_b.reshape(1, 1),
        pred_W1[0:_L, :], pred_W1[_L:2 * _L, :],
        pred_W1[2 * _L:3 * _L, :], pred_W1[3 * _L:, :],
        pred_b1.reshape(1, _L),
        pred_W2, pred_b2.reshape(1, 1),
    )
    out = pl.pallas_call(
        _prims_kernel,
        out_shape=jax.ShapeDtypeStruct((_N * _N, 1), jnp.float32),
    )(*args)
    return out.reshape(_N, _N)
